# Initial kernel scaffold; baseline (speedup 1.0000x reference)
import jax
import jax.numpy as jnp
from jax.experimental import pallas as pl


def kernel(x, edge_index, batch, W1, as1, ad1, b1, W2, as2, ad2, b2, Wp, bp, Wv, bv):
    raise NotImplementedError("write your pallas kernel here")


# speedup vs baseline: 199.7214x; 199.7214x over previous
import functools

import jax
import jax.numpy as jnp
from jax import lax
from jax.experimental import pallas as pl
from jax.experimental.pallas import tpu as pltpu
from jax.experimental.pallas import tpu_sc as plsc

N = 83968
E = 1343488
NPG = 41
LAT = 41

NC, NS, LANES = 2, 16, 16
NW = NC * NS
EPW = E // NW
JW = 128
CH = 8
K = CH * JW
NCHUNK = EPW // K
RPT = N // NS
GROUPS = K // LANES

RB = 2048
NRB = N // RB


def _sc_pass1():
    mesh = plsc.VectorSubcoreMesh(
        core_axis_name="c", subcore_axis_name="s", num_cores=NC, num_subcores=NS
    )
    NACC = 5
    DRAIN = NACC * CH * JW

    @functools.partial(
        pl.kernel,
        mesh=mesh,
        out_type=jax.ShapeDtypeStruct((NC * NACC * N,), jnp.float32),
        scratch_types=[
            pltpu.VMEM((CH, JW), jnp.int32),
            pltpu.VMEM((CH, JW), jnp.int32),
            [pltpu.VMEM((K,), jnp.float32) for _ in range(6)],
            [pltpu.VMEM((K,), jnp.float32) for _ in range(NACC)],
            pltpu.VMEM((DRAIN,), jnp.float32),
            [pltpu.VMEM_SHARED((N,), jnp.float32) for _ in range(NACC)],
            [pltpu.SemaphoreType.DMA for _ in range(7)],
        ],
    )
    def sc1(src_hbm, dst_hbm, t_as, t_ad, t0, t1, t2, t3, zn_hbm, out_hbm,
            idx_s, idx_d, gat, prod, drain, accs, sems):
        cid = lax.axis_index("c")
        sid = lax.axis_index("s")
        wid = sid * NC + cid
        for t in range(NACC):
            pltpu.sync_copy(zn_hbm.at[pl.ds(sid * RPT, RPT)],
                            accs[t].at[pl.ds(sid * RPT, RPT)])
        plsc.subcore_barrier()
        row0 = wid * (EPW // JW)
        vas, vad, v0, v1, v2, v3 = gat
        oe, o0, o1, o2, o3 = prod
        addsem = sems[6]

        def chunk(i, carry):
            base = row0 + i * CH
            pltpu.sync_copy(src_hbm.at[pl.ds(base, CH)], idx_s)
            pltpu.sync_copy(dst_hbm.at[pl.ds(base, CH)], idx_d)
            cps = []
            for j in range(CH):
                sl = pl.ds(j * JW, JW)
                cps.append(pltpu.async_copy(t_as.at[idx_s.at[j]], vas.at[sl], sems[0]))
                cps.append(pltpu.async_copy(t_ad.at[idx_d.at[j]], vad.at[sl], sems[1]))
                cps.append(pltpu.async_copy(t0.at[idx_s.at[j]], v0.at[sl], sems[2]))
                cps.append(pltpu.async_copy(t1.at[idx_s.at[j]], v1.at[sl], sems[3]))
                cps.append(pltpu.async_copy(t2.at[idx_s.at[j]], v2.at[sl], sems[4]))
                cps.append(pltpu.async_copy(t3.at[idx_s.at[j]], v3.at[sl], sems[5]))

            @pl.when(i > 0)
            def _():
                pltpu.make_async_copy(zn_hbm.at[pl.ds(0, DRAIN)], drain,
                                      addsem).wait()

            for cp in cps:
                cp.wait()

            def grp(g, c2):
                sl = pl.ds(g * LANES, LANES)
                e = vas[sl] + vad[sl]
                e = jnp.where(e > 0.0, e, e * jnp.float32(0.2))
                ex = jnp.exp(e)
                oe[sl] = ex
                o0[sl] = ex * v0[sl]
                o1[sl] = ex * v1[sl]
                o2[sl] = ex * v2[sl]
                o3[sl] = ex * v3[sl]
                return c2

            lax.fori_loop(0, GROUPS, grp, 0)
            for j in range(CH):
                sl = pl.ds(j * JW, JW)
                pltpu.async_copy(oe.at[sl], accs[0].at[idx_d.at[j]], addsem, add=True)
                pltpu.async_copy(o0.at[sl], accs[1].at[idx_d.at[j]], addsem, add=True)
                pltpu.async_copy(o1.at[sl], accs[2].at[idx_d.at[j]], addsem, add=True)
                pltpu.async_copy(o2.at[sl], accs[3].at[idx_d.at[j]], addsem, add=True)
                pltpu.async_copy(o3.at[sl], accs[4].at[idx_d.at[j]], addsem, add=True)
            return carry

        lax.fori_loop(0, NCHUNK, chunk, 0)
        pltpu.make_async_copy(zn_hbm.at[pl.ds(0, DRAIN)], drain, addsem).wait()
        plsc.subcore_barrier()
        for t in range(NACC):
            off = (cid * NACC + t) * N + sid * RPT
            pltpu.sync_copy(accs[t].at[pl.ds(sid * RPT, RPT)],
                            out_hbm.at[pl.ds(off, RPT)])

    return sc1


def _sc_pass2():
    mesh = plsc.VectorSubcoreMesh(
        core_axis_name="c", subcore_axis_name="s", num_cores=NC, num_subcores=NS
    )
    NACC = 2
    DRAIN = NACC * CH * JW

    @functools.partial(
        pl.kernel,
        mesh=mesh,
        out_type=jax.ShapeDtypeStruct((NC * NACC * N,), jnp.float32),
        scratch_types=[
            pltpu.VMEM((CH, JW), jnp.int32),
            pltpu.VMEM((CH, JW), jnp.int32),
            [pltpu.VMEM((K,), jnp.float32) for _ in range(2)],
            [pltpu.VMEM((K,), jnp.float32) for _ in range(NACC)],
            pltpu.VMEM((16,), jnp.float32),
            pltpu.VMEM((16,), jnp.float32),
            pltpu.VMEM((DRAIN,), jnp.float32),
            [pltpu.VMEM_SHARED((N,), jnp.float32) for _ in range(NACC)],
            [pltpu.SemaphoreType.DMA for _ in range(3)],
        ],
    )
    def sc2(src_hbm, dst_hbm, t_c, as2_hbm, ad2_hbm, zn_hbm, out_hbm,
            idx_s, idx_d, gat, prod, as2v, ad2v, drain, accs, sems):
        cid = lax.axis_index("c")
        sid = lax.axis_index("s")
        wid = sid * NC + cid
        for t in range(NACC):
            pltpu.sync_copy(zn_hbm.at[pl.ds(sid * RPT, RPT)],
                            accs[t].at[pl.ds(sid * RPT, RPT)])
        pltpu.sync_copy(as2_hbm, as2v)
        pltpu.sync_copy(ad2_hbm, ad2v)
        plsc.subcore_barrier()
        row0 = wid * (EPW // JW)
        vcs, vcd = gat
        oe, o0 = prod
        addsem = sems[2]
        A = as2v[...]
        B = ad2v[...]

        def chunk(i, carry):
            base = row0 + i * CH
            pltpu.sync_copy(src_hbm.at[pl.ds(base, CH)], idx_s)
            pltpu.sync_copy(dst_hbm.at[pl.ds(base, CH)], idx_d)
            cps = []
            for j in range(CH):
                sl = pl.ds(j * JW, JW)
                cps.append(pltpu.async_copy(t_c.at[idx_s.at[j]], vcs.at[sl], sems[0]))
                cps.append(pltpu.async_copy(t_c.at[idx_d.at[j]], vcd.at[sl], sems[1]))

            @pl.when(i > 0)
            def _():
                pltpu.make_async_copy(zn_hbm.at[pl.ds(0, DRAIN)], drain,
                                      addsem).wait()

            for cp in cps:
                cp.wait()

            def grp(g, c2):
                sl = pl.ds(g * LANES, LANES)
                cs = vcs[sl]
                e = A * cs + B * vcd[sl]
                e = jnp.where(e > 0.0, e, e * jnp.float32(0.2))
                ex = jnp.exp(e)
                oe[sl] = ex
                o0[sl] = ex * cs
                return c2

            lax.fori_loop(0, GROUPS, grp, 0)
            for j in range(CH):
                sl = pl.ds(j * JW, JW)
                pltpu.async_copy(oe.at[sl], accs[0].at[idx_d.at[j]], addsem, add=True)
                pltpu.async_copy(o0.at[sl], accs[1].at[idx_d.at[j]], addsem, add=True)
            return carry

        lax.fori_loop(0, NCHUNK, chunk, 0)
        pltpu.make_async_copy(zn_hbm.at[pl.ds(0, DRAIN)], drain, addsem).wait()
        plsc.subcore_barrier()
        for t in range(NACC):
            off = (cid * NACC + t) * N + sid * RPT
            pltpu.sync_copy(accs[t].at[pl.ds(sid * RPT, RPT)],
                            out_hbm.at[pl.ds(off, RPT)])

    return sc2


def _tc_a(xt_ref, vs_ref, vd_ref, as_ref, ad_ref):
    xt = xt_ref[...]
    as_ref[...] = jnp.dot(vs_ref[...], xt, preferred_element_type=jnp.float32, precision=lax.Precision.HIGHEST)
    ad_ref[...] = jnp.dot(vd_ref[...], xt, preferred_element_type=jnp.float32, precision=lax.Precision.HIGHEST)


def _tc_b(acc_ref, xt_ref, as_ref, ad_ref, w1t_ref, b1_ref, w2t_ref,
          c_ref):
    acc = acc_ref[...]
    xt = xt_ref[...]
    es = as_ref[...] + ad_ref[...]
    es = jnp.where(es > 0.0, es, es * 0.2)
    exs = jnp.exp(es)
    den = acc[0:1] + acc[5:6] + exs
    s = acc[1:5] + acc[6:10] + exs * xt
    out1 = jnp.dot(w1t_ref[...], s, preferred_element_type=jnp.float32, precision=lax.Precision.HIGHEST)
    out1 = out1 / (den + 1e-16) + b1_ref[...]
    h = jnp.maximum(out1, 0.0)
    c_ref[...] = jnp.dot(w2t_ref[...], h, preferred_element_type=jnp.float32, precision=lax.Precision.HIGHEST)


def _tc_c(acc_ref, c_ref, as2_ref, ad2_ref, b2_ref, o_ref):
    acc = acc_ref[...]
    c = c_ref[...]
    es = (as2_ref[0] + ad2_ref[0]) * c
    es = jnp.where(es > 0.0, es, es * 0.2)
    exs = jnp.exp(es)
    den = acc[0:1] + acc[2:3] + exs
    num = acc[1:2] + acc[3:4] + exs * c
    o_ref[...] = jnp.maximum(num / (den + 1e-16) + b2_ref[0], 0.0)


def _tc_d(sh_ref, wp_ref, bp_ref, wv_ref, bv_ref, pi_ref, vf_ref):
    sh = sh_ref[...]
    pi_ref[...] = jnp.maximum(
        jnp.dot(sh, wp_ref[...], preferred_element_type=jnp.float32, precision=lax.Precision.HIGHEST)
        + bp_ref[...], 0.0)
    vf_ref[...] = jnp.maximum(
        jnp.dot(sh, wv_ref[...], preferred_element_type=jnp.float32, precision=lax.Precision.HIGHEST)
        + bv_ref[...], 0.0)


_SC1 = _sc_pass1()
_SC2 = _sc_pass2()


def kernel(x, edge_index, batch, W1, as1, ad1, b1, W2, as2, ad2, b2,
           Wp, bp, Wv, bv):
    f32 = jnp.float32
    src2d = edge_index[0].reshape(E // JW, JW)
    dst2d = edge_index[1].reshape(E // JW, JW)
    xt = x.T
    vs = jnp.dot(W1, as1, precision=lax.Precision.HIGHEST).reshape(1, 4)
    vd = jnp.dot(W1, ad1, precision=lax.Precision.HIGHEST).reshape(1, 4)
    zn = jnp.zeros((N,), f32)

    asv, adv = pl.pallas_call(
        _tc_a,
        grid=(NRB,),
        in_specs=[
            pl.BlockSpec((4, RB), lambda i: (0, i)),
            pl.BlockSpec((1, 4), lambda i: (0, 0)),
            pl.BlockSpec((1, 4), lambda i: (0, 0)),
        ],
        out_specs=[
            pl.BlockSpec((1, RB), lambda i: (0, i)),
            pl.BlockSpec((1, RB), lambda i: (0, i)),
        ],
        out_shape=[
            jax.ShapeDtypeStruct((1, N), f32),
            jax.ShapeDtypeStruct((1, N), f32),
        ],
    )(xt, vs, vd)

    acc1 = _SC1(src2d, dst2d, asv.reshape(N), adv.reshape(N),
                xt[0], xt[1], xt[2], xt[3], zn)

    c = pl.pallas_call(
        _tc_b,
        grid=(NRB,),
        in_specs=[
            pl.BlockSpec((10, RB), lambda i: (0, i)),
            pl.BlockSpec((4, RB), lambda i: (0, i)),
            pl.BlockSpec((1, RB), lambda i: (0, i)),
            pl.BlockSpec((1, RB), lambda i: (0, i)),
            pl.BlockSpec((LAT, 4), lambda i: (0, 0)),
            pl.BlockSpec((LAT, 1), lambda i: (0, 0)),
            pl.BlockSpec((1, LAT), lambda i: (0, 0)),
        ],
        out_specs=pl.BlockSpec((1, RB), lambda i: (0, i)),
        out_shape=jax.ShapeDtypeStruct((1, N), f32),
    )(acc1.reshape(NC * 5, N), xt, asv, adv, W1.T, b1.reshape(LAT, 1), W2.T)

    acc2 = _SC2(src2d, dst2d, c.reshape(N),
                jnp.broadcast_to(as2, (16,)).astype(f32),
                jnp.broadcast_to(ad2, (16,)).astype(f32), zn)

    out2 = pl.pallas_call(
        _tc_c,
        grid=(NRB,),
        in_specs=[
            pl.BlockSpec((4, RB), lambda i: (0, i)),
            pl.BlockSpec((1, RB), lambda i: (0, i)),
            pl.BlockSpec(memory_space=pltpu.SMEM),
            pl.BlockSpec(memory_space=pltpu.SMEM),
            pl.BlockSpec(memory_space=pltpu.SMEM),
        ],
        out_specs=pl.BlockSpec((1, RB), lambda i: (0, i)),
        out_shape=jax.ShapeDtypeStruct((1, N), f32),
    )(acc2.reshape(NC * 2, N), c, as2, ad2, b2)

    shared = out2.reshape(N // NPG, NPG)

    pi, vf = pl.pallas_call(
        _tc_d,
        in_specs=[
            pl.BlockSpec((N // NPG, NPG), lambda: (0, 0)),
            pl.BlockSpec((LAT, LAT), lambda: (0, 0)),
            pl.BlockSpec((1, LAT), lambda: (0, 0)),
            pl.BlockSpec((LAT, LAT), lambda: (0, 0)),
            pl.BlockSpec((1, LAT), lambda: (0, 0)),
        ],
        out_specs=[
            pl.BlockSpec((N // NPG, NPG), lambda: (0, 0)),
            pl.BlockSpec((N // NPG, NPG), lambda: (0, 0)),
        ],
        out_shape=[
            jax.ShapeDtypeStruct((N // NPG, NPG), f32),
            jax.ShapeDtypeStruct((N // NPG, NPG), f32),
        ],
    )(shared, Wp, bp.reshape(1, LAT), Wv, bv.reshape(1, LAT))

    return (pi, vf)
